# baseline (speedup 1.0000x reference)
import functools

import jax
import jax.numpy as jnp
from jax import lax
from jax.experimental import pallas as pl
from jax.experimental.pallas import tpu as pltpu
from jax.experimental.pallas import tpu_sc as plsc

_N = 10000
_NP = 10240
_NC = 2
_NS = 16
_NW = _NC * _NS
_ECH = 16384


def _sc_mesh():
    return plsc.VectorSubcoreMesh(
        core_axis_name="c", subcore_axis_name="s",
        num_cores=_NC, num_subcores=_NS)

_SC_PARAMS = dict(
    compiler_params=pltpu.CompilerParams(
        needs_layout_passes=False, use_tc_tiling_on_sc=False))


@functools.lru_cache(maxsize=None)
def _deg_kernel(kc):
    @functools.partial(
        pl.kernel,
        out_type=jax.ShapeDtypeStruct((_NW, _NP), jnp.float32),
        mesh=_sc_mesh(),
        scratch_types=[
            pltpu.VMEM((kc, 128), jnp.int32),
            pltpu.VMEM((_NP,), jnp.float32),
        ],
        **_SC_PARAMS,
    )
    def deg_k(dst_hbm, out_hbm, dst_v, deg_v):
        cid = lax.axis_index("c")
        sid = lax.axis_index("s")
        wid = cid * _NS + sid
        pltpu.sync_copy(dst_hbm.at[wid], dst_v)
        zeros16 = jnp.zeros((16,), jnp.float32)

        @pl.loop(0, _NP // 16)
        def _(i):
            deg_v[pl.ds(i * 16, 16)] = zeros16

        ones16 = jnp.ones((16,), jnp.float32)

        @pl.loop(0, kc)
        def _(j):
            for c in range(128 // 16):
                idx = dst_v[j, pl.ds(c * 16, 16)]
                plsc.addupdate_scatter(deg_v, [idx], ones16)

        pltpu.sync_copy(deg_v, out_hbm.at[wid])

    return deg_k


@functools.lru_cache(maxsize=None)
def _col_scatter_kernel(feat, nch):
    cpt = feat // _NW
    assert cpt * _NW == feat and nch % 2 == 0

    @functools.partial(
        pl.kernel,
        out_type=jax.ShapeDtypeStruct((feat, _NP), jnp.float32),
        mesh=_sc_mesh(),
        scratch_types=(
            [pltpu.VMEM((_NP,), jnp.float32) for _ in range(cpt)]
            + [pltpu.VMEM((_NP,), jnp.float32) for _ in range(cpt)]
            + [pltpu.VMEM((_ECH,), jnp.int32) for _ in range(4)]
            + [pltpu.SemaphoreType.DMA, pltpu.SemaphoreType.DMA]
        ),
        **_SC_PARAMS,
    )
    def scat_k(tab_hbm, src_hbm, dst_hbm, out_hbm, *bufs):
        cols = bufs[:cpt]
        accs = bufs[cpt:2 * cpt]
        s0, d0, s1, d1 = bufs[2 * cpt:2 * cpt + 4]
        sem_a, sem_b = bufs[2 * cpt + 4:]
        cid = lax.axis_index("c")
        sid = lax.axis_index("s")
        wid = cid * _NS + sid
        zeros16 = jnp.zeros((16,), jnp.float32)

        for k in range(cpt):
            pltpu.sync_copy(tab_hbm.at[wid * cpt + k], cols[k])

        @pl.loop(0, _NP // 16)
        def _(i):
            for k in range(cpt):
                accs[k][pl.ds(i * 16, 16)] = zeros16

        pltpu.async_copy(src_hbm.at[0], s0, sem_a)
        pltpu.async_copy(dst_hbm.at[0], d0, sem_a)

        def process(sbuf, dbuf):
            @pl.loop(0, _ECH // 16, unroll=4)
            def _(i):
                s = sbuf[pl.ds(i * 16, 16)]
                d = dbuf[pl.ds(i * 16, 16)]
                for k in range(cpt):
                    v = plsc.load_gather(cols[k], [s])
                    plsc.addupdate_scatter(accs[k], [d], v)

        @pl.loop(0, nch, step=2)
        def _(c):
            pltpu.async_copy(src_hbm.at[c + 1], s1, sem_b)
            pltpu.async_copy(dst_hbm.at[c + 1], d1, sem_b)
            pltpu.make_async_copy(src_hbm.at[c], s0, sem_a).wait()
            pltpu.make_async_copy(dst_hbm.at[c], d0, sem_a).wait()
            process(s0, d0)

            @pl.when(c + 2 < nch)
            def _():
                pltpu.async_copy(src_hbm.at[c + 2], s0, sem_a)
                pltpu.async_copy(dst_hbm.at[c + 2], d0, sem_a)

            pltpu.make_async_copy(src_hbm.at[c + 1], s1, sem_b).wait()
            pltpu.make_async_copy(dst_hbm.at[c + 1], d1, sem_b).wait()
            process(s1, d1)

        for k in range(cpt):
            pltpu.sync_copy(accs[k], out_hbm.at[wid * cpt + k])

    return scat_k


def _tc_layer1(x_pad, deg_parts, w1):
    def body(x_ref, dp_ref, w_ref, y_ref, dinv_ref):
        deg = 1.0 + jnp.sum(dp_ref[...], axis=0, keepdims=True)
        dinv = lax.rsqrt(deg)
        xw_t = lax.dot_general(
            w_ref[...], x_ref[...], (((0,), (1,)), ((), ())),
            preferred_element_type=jnp.float32)
        y_ref[...] = xw_t * dinv
        dinv_ref[...] = dinv

    return pl.pallas_call(
        body,
        out_shape=[
            jax.ShapeDtypeStruct((32, _NP), jnp.float32),
            jax.ShapeDtypeStruct((1, _NP), jnp.float32),
        ],
    )(x_pad, deg_parts, w1)


def _tc_layer2(agg1, y1, dinv, b1_col, scale_t, w2):
    def body(a_ref, y1_ref, dinv_ref, b_ref, s_ref, w_ref, y2_ref):
        dinv = dinv_ref[...]
        h = (a_ref[...] + y1_ref[...]) * dinv + b_ref[...]
        h = jnp.maximum(h, 0.0) * s_ref[...]
        y2_ref[...] = lax.dot_general(
            w_ref[...], h, (((0,), (0,)), ((), ())),
            preferred_element_type=jnp.float32) * dinv

    return pl.pallas_call(
        body,
        out_shape=jax.ShapeDtypeStruct((64, _NP), jnp.float32),
    )(agg1, y1, dinv, b1_col, scale_t, w2)


def _tc_layer3(agg2, y2, dinv, b2_col):
    def body(a_ref, y2_ref, dinv_ref, b_ref, o_ref):
        z = (a_ref[...] + y2_ref[...]) * dinv_ref[...] + b_ref[...]
        o_ref[...] = jnp.where(z > 0, z, 0.01 * z)

    return pl.pallas_call(
        body,
        out_shape=jax.ShapeDtypeStruct((64, _NP), jnp.float32),
    )(agg2, y2, dinv, b2_col)


def kernel(x, edge_index, W1, b1, W2, b2):
    n = x.shape[0]
    e = edge_index.shape[1]
    nch = -(-e // _ECH)
    nch = -(-nch // 2) * 2
    e_pad = nch * _ECH
    kc = e_pad // (_NW * 128)

    fill = jnp.full((e_pad - e,), _N, jnp.int32)
    src_ch = jnp.concatenate([edge_index[0], fill]).reshape(nch, _ECH)
    dst_ch = jnp.concatenate([edge_index[1], fill]).reshape(nch, _ECH)
    dst_tile = dst_ch.reshape(_NW, kc, 128)
    x_pad = jnp.pad(x, ((0, _NP - n), (0, 0)))
    mask = jax.random.bernoulli(jax.random.key(42), 0.5, (n, W1.shape[1]))
    scale_t = jnp.pad(jnp.where(mask, 2.0, 0.0).astype(jnp.float32).T,
                      ((0, 0), (0, _NP - n)))

    degp = _deg_kernel(kc)(dst_tile)
    y1, dinv = _tc_layer1(x_pad, degp, W1)
    agg1 = _col_scatter_kernel(32, nch)(y1, src_ch, dst_ch)
    y2 = _tc_layer2(agg1, y1, dinv, b1.reshape(-1, 1), scale_t, W2)
    agg2 = _col_scatter_kernel(64, nch)(y2, src_ch, dst_ch)
    out_t = _tc_layer3(agg2, y2, dinv, b2.reshape(-1, 1))
    return out_t.T[:n]


# speedup vs baseline: 1.1919x; 1.1919x over previous
import functools

import jax
import jax.numpy as jnp
from jax import lax
from jax.experimental import pallas as pl
from jax.experimental.pallas import tpu as pltpu
from jax.experimental.pallas import tpu_sc as plsc

_N = 10000
_NP = 10240
_NC = 2
_NS = 16
_NW = _NC * _NS
_CH = 128
_GP = 4


def _sc_mesh():
    return plsc.VectorSubcoreMesh(
        core_axis_name="c", subcore_axis_name="s",
        num_cores=_NC, num_subcores=_NS)

_SC_PARAMS = dict(
    compiler_params=pltpu.CompilerParams(
        needs_layout_passes=False, use_tc_tiling_on_sc=False))


@functools.lru_cache(maxsize=None)
def _deg_kernel(kc):
    @functools.partial(
        pl.kernel,
        out_type=jax.ShapeDtypeStruct((_NW, _NP), jnp.float32),
        mesh=_sc_mesh(),
        scratch_types=[
            pltpu.VMEM((kc, _CH), jnp.int32),
            pltpu.VMEM((_NP,), jnp.float32),
        ],
        **_SC_PARAMS,
    )
    def deg_k(dst_hbm, out_hbm, dst_v, deg_v):
        cid = lax.axis_index("c")
        sid = lax.axis_index("s")
        wid = cid * _NS + sid
        pltpu.sync_copy(dst_hbm.at[wid], dst_v)
        zeros16 = jnp.zeros((16,), jnp.float32)

        @pl.loop(0, _NP // 16)
        def _(i):
            deg_v[pl.ds(i * 16, 16)] = zeros16

        ones16 = jnp.ones((16,), jnp.float32)

        @pl.loop(0, kc)
        def _(j):
            for c in range(_CH // 16):
                idx = dst_v[j, pl.ds(c * 16, 16)]
                plsc.addupdate_scatter(deg_v, [idx], ones16)

        pltpu.sync_copy(deg_v, out_hbm.at[wid])

    return deg_k


@functools.lru_cache(maxsize=None)
def _scatter_kernel(feat, kc):
    rpt = _NP // _NS
    gr = _GP * _CH

    @functools.partial(
        pl.kernel,
        out_type=jax.ShapeDtypeStruct((_NC, _NP, feat), jnp.float32),
        mesh=_sc_mesh(),
        scratch_types=[
            pltpu.VMEM((kc, _CH), jnp.int32),
            pltpu.VMEM((kc, _CH), jnp.int32),
            pltpu.VMEM((gr, feat), jnp.float32),
            pltpu.VMEM_SHARED((_NP, feat), jnp.float32),
            pltpu.SemaphoreType.DMA,
            pltpu.SemaphoreType.DMA,
        ],
        **_SC_PARAMS,
    )
    def scat_k(tab_hbm, src_hbm, dst_hbm, out_hbm,
               src_v, dst_v, rows_v, acc, gsem, ssem):
        cid = lax.axis_index("c")
        sid = lax.axis_index("s")
        wid = cid * _NS + sid
        zeros16 = jnp.zeros((16,), jnp.float32)

        @pl.loop(0, gr)
        def _(r):
            for c in range(feat // 16):
                rows_v[r, pl.ds(c * 16, 16)] = zeros16

        base = sid * rpt
        pltpu.sync_copy(rows_v, acc.at[pl.ds(base, gr)])
        pltpu.sync_copy(rows_v.at[pl.ds(0, rpt - gr)],
                        acc.at[pl.ds(base + gr, rpt - gr)])
        pltpu.sync_copy(src_hbm.at[wid], src_v)
        pltpu.sync_copy(dst_hbm.at[wid], dst_v)
        plsc.subcore_barrier()

        @pl.loop(0, kc // _GP)
        def _(g):
            gds = [
                pltpu.async_copy(
                    tab_hbm.at[src_v.at[g * _GP + q]],
                    rows_v.at[pl.ds(q * _CH, _CH)], gsem)
                for q in range(_GP)
            ]
            sds = []
            for q in range(_GP):
                gds[q].wait()
                sds.append(pltpu.async_copy(
                    rows_v.at[pl.ds(q * _CH, _CH)],
                    acc.at[dst_v.at[g * _GP + q]], ssem, add=True))
            for d in sds:
                d.wait()

        plsc.subcore_barrier()
        pltpu.sync_copy(acc.at[pl.ds(base, rpt)],
                        out_hbm.at[cid, pl.ds(base, rpt)])

    return scat_k


def _tc_layer1(x_pad, deg_col, w1):
    def body(x_ref, d_ref, w_ref, y_ref, dinv_ref):
        dinv = lax.rsqrt(d_ref[...])
        xw = jnp.dot(x_ref[...], w_ref[...],
                     preferred_element_type=jnp.float32)
        y_ref[...] = xw * dinv
        dinv_ref[...] = dinv

    return pl.pallas_call(
        body,
        out_shape=[
            jax.ShapeDtypeStruct((_NP, 32), jnp.float32),
            jax.ShapeDtypeStruct((_NP, 1), jnp.float32),
        ],
    )(x_pad, deg_col, w1)


def _tc_layer2(acc1, y1, dinv, b1_row, scale, w2):
    def body(a_ref, y1_ref, dinv_ref, b_ref, s_ref, w_ref, y2_ref):
        dinv = dinv_ref[...]
        agg = a_ref[0] + a_ref[1] + y1_ref[...]
        h = agg * dinv + b_ref[...]
        h = jnp.maximum(h, 0.0) * s_ref[...]
        y2_ref[...] = jnp.dot(h, w_ref[...],
                              preferred_element_type=jnp.float32) * dinv

    return pl.pallas_call(
        body,
        out_shape=jax.ShapeDtypeStruct((_NP, 64), jnp.float32),
    )(acc1, y1, dinv, b1_row, scale, w2)


def _tc_layer3(acc2, y2, dinv, b2_row):
    def body(a_ref, y2_ref, dinv_ref, b_ref, o_ref):
        z = (a_ref[0] + a_ref[1] + y2_ref[...]) * dinv_ref[...] + b_ref[...]
        o_ref[...] = jnp.where(z > 0, z, 0.01 * z)

    return pl.pallas_call(
        body,
        out_shape=jax.ShapeDtypeStruct((_NP, 64), jnp.float32),
    )(acc2, y2, dinv, b2_row)


def kernel(x, edge_index, W1, b1, W2, b2):
    n = x.shape[0]
    e = edge_index.shape[1]
    kc = -(-e // (_NW * _CH))
    kc = -(-kc // _GP) * _GP
    e_pad = _NW * kc * _CH

    fill = jnp.full((e_pad - e,), _N, jnp.int32)
    srcp = jnp.concatenate([edge_index[0], fill]).reshape(_NW, kc, _CH)
    dstp = jnp.concatenate([edge_index[1], fill]).reshape(_NW, kc, _CH)
    x_pad = jnp.pad(x, ((0, _NP - n), (0, 0)))
    mask = jax.random.bernoulli(jax.random.key(42), 0.5, (n, W1.shape[1]))
    scale = jnp.pad(jnp.where(mask, 2.0, 0.0).astype(jnp.float32),
                    ((0, _NP - n), (0, 0)))

    degp = _deg_kernel(kc)(dstp)
    deg_col = 1.0 + jnp.sum(degp, axis=0)[:, None]
    y1, dinv = _tc_layer1(x_pad, deg_col, W1)
    acc1 = _scatter_kernel(32, kc)(y1, srcp, dstp)
    y2 = _tc_layer2(acc1, y1, dinv, b1.reshape(1, -1), scale, W2)
    acc2 = _scatter_kernel(64, kc)(y2, srcp, dstp)
    out = _tc_layer3(acc2, y2, dinv, b2.reshape(1, -1))
    return out[:n]
